# baseline (speedup 1.0000x reference)
import functools

import jax
import jax.numpy as jnp
from jax import lax
from jax.experimental import pallas as pl
from jax.experimental.pallas import tpu as pltpu
from jax.experimental.pallas import tpu_sc as plsc

VOCAB_DIM = 64
NC, NS = 2, 16
NW = NC * NS
IDXW = 128
K = 4
CHUNK = K * IDXW


def _make_sc_gather(n_rows: int):
    b_per_w = n_rows // NW
    n_chunks = b_per_w // CHUNK
    idx_rows_per_w = b_per_w // IDXW

    mesh = plsc.VectorSubcoreMesh(core_axis_name="c", subcore_axis_name="s")

    @functools.partial(
        pl.kernel,
        mesh=mesh,
        out_type=jax.ShapeDtypeStruct((n_rows, VOCAB_DIM), jnp.float32),
        compiler_params=pltpu.CompilerParams(use_tc_tiling_on_sc=False),
        scratch_types=[
            pltpu.VMEM((K, IDXW), jnp.int32),
            pltpu.VMEM((CHUNK, VOCAB_DIM), jnp.float32),
            pltpu.SemaphoreType.DMA,
        ],
    )
    def sc_gather(ids_hbm, table_hbm, out_hbm, idx_v, rows_v, sem):
        wid = lax.axis_index("s") * NC + lax.axis_index("c")
        idx_row0 = wid * idx_rows_per_w
        base0 = wid * b_per_w

        def body(i, _):
            pltpu.sync_copy(ids_hbm.at[pl.ds(idx_row0 + i * K, K), :], idx_v)
            copies = []
            for j in range(K):
                copies.append(
                    pltpu.async_copy(
                        table_hbm.at[idx_v.at[j]],
                        rows_v.at[pl.ds(j * IDXW, IDXW), :],
                        sem,
                    )
                )
            for c in copies:
                c.wait()
            pltpu.sync_copy(rows_v, out_hbm.at[pl.ds(base0 + i * CHUNK, CHUNK), :])
            return 0

        lax.fori_loop(0, n_chunks, body, 0)

    return sc_gather


def kernel(input_ids, table):
    batch, seq = input_ids.shape
    n_rows = batch * seq
    ids = input_ids.reshape(n_rows // IDXW, IDXW).astype(jnp.int32)
    out = _make_sc_gather(n_rows)(ids, table)
    return out.reshape(batch, seq, VOCAB_DIM)


# speedup vs baseline: 1.0347x; 1.0347x over previous
import functools

import jax
import jax.numpy as jnp
from jax import lax
from jax.experimental import pallas as pl
from jax.experimental.pallas import tpu as pltpu
from jax.experimental.pallas import tpu_sc as plsc

VOCAB_DIM = 64
NC, NS = 2, 16
NW = NC * NS
IDXW = 128
K = 5
CHUNK = K * IDXW
NBUF = 2


def _make_sc_gather(n_rows: int):
    b_per_w = n_rows // NW
    n_chunks = b_per_w // CHUNK
    n_pairs = n_chunks // NBUF
    idx_rows_per_w = b_per_w // IDXW

    mesh = plsc.VectorSubcoreMesh(core_axis_name="c", subcore_axis_name="s")

    @functools.partial(
        pl.kernel,
        mesh=mesh,
        out_type=jax.ShapeDtypeStruct((n_rows, VOCAB_DIM), jnp.float32),
        compiler_params=pltpu.CompilerParams(use_tc_tiling_on_sc=False),
        scratch_types=[
            pltpu.VMEM((NBUF, K, IDXW), jnp.int32),
            pltpu.VMEM((NBUF, CHUNK, VOCAB_DIM), jnp.float32),
            pltpu.SemaphoreType.DMA,
            pltpu.SemaphoreType.DMA,
            pltpu.SemaphoreType.DMA,
        ],
    )
    def sc_gather(ids_hbm, table_hbm, out_hbm, idx_v, rows_v, gsem, wsem0, wsem1):
        wid = lax.axis_index("s") * NC + lax.axis_index("c")
        idx_row0 = wid * idx_rows_per_w
        base0 = wid * b_per_w
        wsems = (wsem0, wsem1)

        def chunk(i, b, wait_writeback):
            rows_b = rows_v.at[b]
            idx_b = idx_v.at[b]
            if wait_writeback:
                pltpu.make_async_copy(
                    rows_b, out_hbm.at[pl.ds(0, CHUNK), :], wsems[b]
                ).wait()
            pltpu.sync_copy(ids_hbm.at[pl.ds(idx_row0 + i * K, K), :], idx_b)
            copies = [
                pltpu.async_copy(
                    table_hbm.at[idx_b.at[j]],
                    rows_b.at[pl.ds(j * IDXW, IDXW), :],
                    gsem,
                )
                for j in range(K)
            ]
            for c in copies:
                c.wait()
            pltpu.async_copy(
                rows_b, out_hbm.at[pl.ds(base0 + i * CHUNK, CHUNK), :], wsems[b]
            )

        for b in range(NBUF):
            chunk(b, b, wait_writeback=False)

        def pair_body(p, _):
            for b in range(NBUF):
                chunk(p * NBUF + b, b, wait_writeback=True)
            return 0

        lax.fori_loop(1, n_pairs, pair_body, 0)

        for b in range(NBUF):
            pltpu.make_async_copy(
                rows_v.at[b], out_hbm.at[pl.ds(0, CHUNK), :], wsems[b]
            ).wait()

    return sc_gather


def kernel(input_ids, table):
    batch, seq = input_ids.shape
    n_rows = batch * seq
    ids = input_ids.reshape(n_rows // IDXW, IDXW).astype(jnp.int32)
    out = _make_sc_gather(n_rows)(ids, table)
    return out.reshape(batch, seq, VOCAB_DIM)
